# baseline (speedup 1.0000x reference)
import functools

import jax
import jax.numpy as jnp
from jax import lax
from jax.experimental import pallas as pl
from jax.experimental.pallas import tpu as pltpu
from jax.experimental.pallas import tpu_sc as plsc

N_TOK = 16384
NE = 8192
D = 256
BM = 512
BMC = 512

SC_NC = 2
SC_NS = 16
SC_NW = SC_NC * SC_NS
SC_CH = 128


def _prep_body(code_ref, cnt_ref, c2_ref):
    c = code_ref[...]
    n = jnp.sqrt(jnp.sum(c * c, axis=1, keepdims=True))
    cn = c / jnp.maximum(n, 1e-12)
    cnt_ref[...] = cn.T
    cnt = cnt_ref[...]
    c2_ref[...] = jnp.sum(cnt * cnt, axis=0, keepdims=True)


def _argmin_body(z_ref, cnt_ref, c2_ref, idx_ref):
    z = z_ref[...]
    zn = z / jnp.maximum(jnp.sqrt(jnp.sum(z * z, axis=1, keepdims=True)), 1e-12)
    z2n = jnp.sum(zn * zn, axis=1, keepdims=True)
    s2 = lax.dot_general(zn * (-2.0), cnt_ref[...], (((1,), (0,)), ((), ())),
                         preferred_element_type=jnp.float32)
    t = (z2n + c2_ref[...]) + s2
    idx_ref[0, 0, :] = jnp.argmin(t, axis=1).astype(jnp.int32)


def _finalize_body(nbc, z_ref, g_ref, zq_ref, loss_ref):
    i = pl.program_id(0)
    g = g_ref[...]
    n = jnp.sqrt(jnp.sum(g * g, axis=1, keepdims=True))
    zq = g / jnp.maximum(n, 1e-12)
    zq_ref[...] = zq
    dlt = zq - z_ref[...]
    ssq = jnp.sum(dlt * dlt, axis=(0, 1), keepdims=True)
    tot = jnp.where(i == 0, jnp.zeros_like(ssq), loss_ref[...]) + ssq
    loss_ref[...] = jnp.where(i == nbc - 1, tot * (1.25 / (N_TOK * D)), tot)


def _sc_gather_body(bpw, code_hbm, idx_hbm, out_hbm, idx_v, rows_v, sem):
    wid = lax.axis_index("s") * SC_NC + lax.axis_index("c")
    base = wid * bpw
    for c in range(bpw // SC_CH):
        b = base + c * SC_CH
        pltpu.sync_copy(idx_hbm.at[pl.ds(b, SC_CH)], idx_v)
        pltpu.async_copy(code_hbm.at[idx_v], rows_v, sem).wait()
        pltpu.sync_copy(rows_v, out_hbm.at[pl.ds(b, SC_CH)])


@functools.cache
def _sc_gather(m):
    mesh = plsc.VectorSubcoreMesh(core_axis_name="c", subcore_axis_name="s")
    return pl.kernel(
        functools.partial(_sc_gather_body, m // SC_NW),
        out_type=jax.ShapeDtypeStruct((m, D), jnp.float32),
        mesh=mesh,
        scratch_types=[
            pltpu.VMEM((SC_CH,), jnp.int32),
            pltpu.VMEM((SC_CH, D), jnp.float32),
            pltpu.SemaphoreType.DMA,
        ],
    )


def _prep_call(code):
    return pl.pallas_call(
        _prep_body,
        out_specs=[
            pl.BlockSpec((D, NE), lambda: (0, 0)),
            pl.BlockSpec((1, NE), lambda: (0, 0)),
        ],
        out_shape=[
            jax.ShapeDtypeStruct((D, NE), jnp.float32),
            jax.ShapeDtypeStruct((1, NE), jnp.float32),
        ],
    )(code)


def _argmin_call(z_flat, cnt, c2):
    nb = z_flat.shape[0] // BM
    return pl.pallas_call(
        _argmin_body,
        grid=(nb,),
        in_specs=[
            pl.BlockSpec((BM, D), lambda i: (i, 0)),
            pl.BlockSpec((D, NE), lambda i: (0, 0)),
            pl.BlockSpec((1, NE), lambda i: (0, 0)),
        ],
        out_specs=pl.BlockSpec((1, 1, BM), lambda i: (i, 0, 0)),
        out_shape=jax.ShapeDtypeStruct((nb, 1, BM), jnp.int32),
    )(z_flat, cnt, c2)


def _finalize_call(z_flat, gathered):
    nbc = z_flat.shape[0] // BMC
    return pl.pallas_call(
        functools.partial(_finalize_body, nbc),
        grid=(nbc,),
        in_specs=[
            pl.BlockSpec((BMC, D), lambda i: (i, 0)),
            pl.BlockSpec((BMC, D), lambda i: (i, 0)),
        ],
        out_specs=[
            pl.BlockSpec((BMC, D), lambda i: (i, 0)),
            pl.BlockSpec((1, 1), lambda i: (0, 0)),
        ],
        out_shape=[
            jax.ShapeDtypeStruct((z_flat.shape[0], D), jnp.float32),
            jax.ShapeDtypeStruct((1, 1), jnp.float32),
        ],
    )(z_flat, gathered)


def kernel(z, code):
    z_flat = z.reshape(N_TOK, D)
    half = N_TOK // 2
    cnt, c2 = _prep_call(code)
    za, zb = z_flat[:half], z_flat[half:]
    idx_a = _argmin_call(za, cnt, c2).reshape(half)
    g_a = _sc_gather(half)(code, idx_a)
    idx_b = _argmin_call(zb, cnt, c2).reshape(half)
    g_b = _sc_gather(half)(code, idx_b)
    zq_a, loss_a = _finalize_call(za, g_a)
    zq_b, loss_b = _finalize_call(zb, g_b)
    idx = jnp.concatenate([idx_a, idx_b])
    zq = jnp.concatenate([zq_a, zq_b]).reshape(z.shape)
    loss = loss_a[0, 0] + loss_b[0, 0]
    return (zq, loss, (None, None, idx))


# speedup vs baseline: 1.0949x; 1.0949x over previous
import functools

import jax
import jax.numpy as jnp
from jax import lax
from jax.experimental import pallas as pl
from jax.experimental.pallas import tpu as pltpu
from jax.experimental.pallas import tpu_sc as plsc

N_TOK = 16384
NE = 8192
D = 256
BM = 512
BMC = 512

SC_NC = 2
SC_NS = 16
SC_NW = SC_NC * SC_NS
SC_CH = 128


def _prep_body(code_ref, cnt_ref, c2_ref):
    c = code_ref[...]
    n = jnp.sqrt(jnp.sum(c * c, axis=1, keepdims=True))
    cn = c / jnp.maximum(n, 1e-12)
    cnt_ref[...] = cn.T
    cnt = cnt_ref[...]
    c2_ref[...] = jnp.sum(cnt * cnt, axis=0, keepdims=True)


def _argmin_body(z_ref, cnt_ref, c2_ref, idx_ref):
    z = z_ref[...]
    zn = z / jnp.maximum(jnp.sqrt(jnp.sum(z * z, axis=1, keepdims=True)), 1e-12)
    z2n = jnp.sum(zn * zn, axis=1, keepdims=True)
    s2 = lax.dot_general(zn * (-2.0), cnt_ref[...], (((1,), (0,)), ((), ())),
                         preferred_element_type=jnp.float32)
    t = (z2n + c2_ref[...]) + s2
    idx_ref[0, 0, :] = jnp.argmin(t, axis=1).astype(jnp.int32)


def _finalize2_body(nh, z_ref, ga_ref, gb_ref, zq_ref, loss_ref):
    i = pl.program_id(0)
    g = jnp.concatenate([ga_ref[...], gb_ref[...]], axis=0)
    z = jnp.concatenate([z_ref[0], z_ref[1]], axis=0)
    n = jnp.sqrt(jnp.sum(g * g, axis=1, keepdims=True))
    zq = g / jnp.maximum(n, 1e-12)
    zq_ref[0] = zq[:BMC]
    zq_ref[1] = zq[BMC:]
    dlt = zq - z
    ssq = jnp.sum(dlt * dlt, axis=(0, 1), keepdims=True)
    tot = jnp.where(i == 0, jnp.zeros_like(ssq), loss_ref[...]) + ssq
    loss_ref[...] = jnp.where(i == nh - 1, tot * (1.25 / (N_TOK * D)), tot)


def _finalize2_call(z_flat, g_a, g_b):
    half = z_flat.shape[0] // 2
    nh = half // BMC
    z3 = z_flat.reshape(2, half, D)
    zq3, loss11 = pl.pallas_call(
        functools.partial(_finalize2_body, nh),
        grid=(nh,),
        in_specs=[
            pl.BlockSpec((2, BMC, D), lambda i: (0, i, 0)),
            pl.BlockSpec((BMC, D), lambda i: (i, 0)),
            pl.BlockSpec((BMC, D), lambda i: (i, 0)),
        ],
        out_specs=[
            pl.BlockSpec((2, BMC, D), lambda i: (0, i, 0)),
            pl.BlockSpec((1, 1), lambda i: (0, 0)),
        ],
        out_shape=[
            jax.ShapeDtypeStruct((2, half, D), jnp.float32),
            jax.ShapeDtypeStruct((1, 1), jnp.float32),
        ],
    )(z3, g_a, g_b)
    return zq3.reshape(z_flat.shape), loss11


def _finalize_body(nbc, z_ref, g_ref, zq_ref, loss_ref):
    i = pl.program_id(0)
    g = g_ref[...]
    n = jnp.sqrt(jnp.sum(g * g, axis=1, keepdims=True))
    zq = g / jnp.maximum(n, 1e-12)
    zq_ref[...] = zq
    dlt = zq - z_ref[...]
    ssq = jnp.sum(dlt * dlt, axis=(0, 1), keepdims=True)
    tot = jnp.where(i == 0, jnp.zeros_like(ssq), loss_ref[...]) + ssq
    loss_ref[...] = jnp.where(i == nbc - 1, tot * (1.25 / (N_TOK * D)), tot)


def _sc_gather_body(bpw, code_hbm, idx_hbm, out_hbm, idx_v, rows_v, sem):
    wid = lax.axis_index("s") * SC_NC + lax.axis_index("c")
    base = wid * bpw
    for c in range(bpw // SC_CH):
        b = base + c * SC_CH
        pltpu.sync_copy(idx_hbm.at[pl.ds(b, SC_CH)], idx_v)
        pltpu.async_copy(code_hbm.at[idx_v], rows_v, sem).wait()
        pltpu.sync_copy(rows_v, out_hbm.at[pl.ds(b, SC_CH)])


@functools.cache
def _sc_gather(m):
    mesh = plsc.VectorSubcoreMesh(core_axis_name="c", subcore_axis_name="s")
    return pl.kernel(
        functools.partial(_sc_gather_body, m // SC_NW),
        out_type=jax.ShapeDtypeStruct((m, D), jnp.float32),
        mesh=mesh,
        scratch_types=[
            pltpu.VMEM((SC_CH,), jnp.int32),
            pltpu.VMEM((SC_CH, D), jnp.float32),
            pltpu.SemaphoreType.DMA,
        ],
    )


def _prep_call(code):
    return pl.pallas_call(
        _prep_body,
        out_specs=[
            pl.BlockSpec((D, NE), lambda: (0, 0)),
            pl.BlockSpec((1, NE), lambda: (0, 0)),
        ],
        out_shape=[
            jax.ShapeDtypeStruct((D, NE), jnp.float32),
            jax.ShapeDtypeStruct((1, NE), jnp.float32),
        ],
    )(code)


def _argmin_call(z_flat, cnt, c2):
    nb = z_flat.shape[0] // BM
    return pl.pallas_call(
        _argmin_body,
        grid=(nb,),
        in_specs=[
            pl.BlockSpec((BM, D), lambda i: (i, 0)),
            pl.BlockSpec((D, NE), lambda i: (0, 0)),
            pl.BlockSpec((1, NE), lambda i: (0, 0)),
        ],
        out_specs=pl.BlockSpec((1, 1, BM), lambda i: (i, 0, 0)),
        out_shape=jax.ShapeDtypeStruct((nb, 1, BM), jnp.int32),
    )(z_flat, cnt, c2)


def _finalize_call(z_flat, gathered):
    nbc = z_flat.shape[0] // BMC
    return pl.pallas_call(
        functools.partial(_finalize_body, nbc),
        grid=(nbc,),
        in_specs=[
            pl.BlockSpec((BMC, D), lambda i: (i, 0)),
            pl.BlockSpec((BMC, D), lambda i: (i, 0)),
        ],
        out_specs=[
            pl.BlockSpec((BMC, D), lambda i: (i, 0)),
            pl.BlockSpec((1, 1), lambda i: (0, 0)),
        ],
        out_shape=[
            jax.ShapeDtypeStruct((z_flat.shape[0], D), jnp.float32),
            jax.ShapeDtypeStruct((1, 1), jnp.float32),
        ],
    )(z_flat, gathered)


def kernel(z, code):
    z_flat = z.reshape(N_TOK, D)
    half = N_TOK // 2
    cnt, c2 = _prep_call(code)
    za, zb = z_flat[:half], z_flat[half:]
    idx_a = _argmin_call(za, cnt, c2).reshape(half)
    g_a = _sc_gather(half)(code, idx_a)
    idx_b = _argmin_call(zb, cnt, c2).reshape(half)
    g_b = _sc_gather(half)(code, idx_b)
    zq_flat, loss11 = _finalize2_call(z_flat, g_a, g_b)
    idx = jnp.concatenate([idx_a, idx_b])
    return (zq_flat.reshape(z.shape), loss11[0, 0], (None, None, idx))


# speedup vs baseline: 1.1248x; 1.0273x over previous
import functools

import jax
import jax.numpy as jnp
from jax import lax
from jax.experimental import pallas as pl
from jax.experimental.pallas import tpu as pltpu
from jax.experimental.pallas import tpu_sc as plsc

N_TOK = 16384
NE = 8192
D = 256
BM = 512
BMC = 512

SC_NC = 2
SC_NS = 16
SC_NW = SC_NC * SC_NS
SC_CH = 128


def _prep_body(code_ref, cnt_ref, c2_ref):
    c = code_ref[...]
    n = jnp.sqrt(jnp.sum(c * c, axis=1, keepdims=True))
    cn = c / jnp.maximum(n, 1e-12)
    cnt_ref[...] = cn.T
    cnt = cnt_ref[...]
    c2_ref[...] = jnp.sum(cnt * cnt, axis=0, keepdims=True)


def _argmin_body(z_ref, cnt_ref, c2_ref, idx_ref):
    z = z_ref[...]
    zn = z / jnp.maximum(jnp.sqrt(jnp.sum(z * z, axis=1, keepdims=True)), 1e-12)
    z2n = jnp.sum(zn * zn, axis=1, keepdims=True)
    s2 = lax.dot_general(zn * (-2.0), cnt_ref[...], (((1,), (0,)), ((), ())),
                         preferred_element_type=jnp.float32)
    t = (z2n + c2_ref[...]) + s2
    idx_ref[0, 0, :] = jnp.argmin(t, axis=1).astype(jnp.int32)


def _finalize2_body(nh, z_ref, ga_ref, gb_ref, zq_ref, loss_ref):
    i = pl.program_id(0)
    g = jnp.concatenate([ga_ref[...], gb_ref[...]], axis=0)
    z = jnp.concatenate([z_ref[0], z_ref[1]], axis=0)
    n = jnp.sqrt(jnp.sum(g * g, axis=1, keepdims=True))
    zq = g / jnp.maximum(n, 1e-12)
    zq_ref[0] = zq[:BMC]
    zq_ref[1] = zq[BMC:]
    dlt = zq - z
    ssq = jnp.sum(dlt * dlt, axis=(0, 1), keepdims=True)
    tot = jnp.where(i == 0, jnp.zeros_like(ssq), loss_ref[...]) + ssq
    loss_ref[...] = jnp.where(i == nh - 1, tot * (1.25 / (N_TOK * D)), tot)


def _finalize2_call(z_flat, g_a, g_b):
    half = z_flat.shape[0] // 2
    nh = half // BMC
    z3 = z_flat.reshape(2, half, D)
    zq3, loss11 = pl.pallas_call(
        functools.partial(_finalize2_body, nh),
        grid=(nh,),
        in_specs=[
            pl.BlockSpec((2, BMC, D), lambda i: (0, i, 0)),
            pl.BlockSpec((BMC, D), lambda i: (i, 0)),
            pl.BlockSpec((BMC, D), lambda i: (i, 0)),
        ],
        out_specs=[
            pl.BlockSpec((2, BMC, D), lambda i: (0, i, 0)),
            pl.BlockSpec((1, 1), lambda i: (0, 0)),
        ],
        out_shape=[
            jax.ShapeDtypeStruct((2, half, D), jnp.float32),
            jax.ShapeDtypeStruct((1, 1), jnp.float32),
        ],
    )(z3, g_a, g_b)
    return zq3.reshape(z_flat.shape), loss11


def _finalize_body(nbc, z_ref, g_ref, zq_ref, loss_ref):
    i = pl.program_id(0)
    g = g_ref[...]
    n = jnp.sqrt(jnp.sum(g * g, axis=1, keepdims=True))
    zq = g / jnp.maximum(n, 1e-12)
    zq_ref[...] = zq
    dlt = zq - z_ref[...]
    ssq = jnp.sum(dlt * dlt, axis=(0, 1), keepdims=True)
    tot = jnp.where(i == 0, jnp.zeros_like(ssq), loss_ref[...]) + ssq
    loss_ref[...] = jnp.where(i == nbc - 1, tot * (1.25 / (N_TOK * D)), tot)


def _sc_gather_body(bpw, code_hbm, idx_hbm, out_hbm, idx_v, rows_v, sem):
    wid = lax.axis_index("s") * SC_NC + lax.axis_index("c")
    base = wid * bpw
    for c in range(bpw // SC_CH):
        b = base + c * SC_CH
        pltpu.sync_copy(idx_hbm.at[pl.ds(b, SC_CH)], idx_v)
        pltpu.async_copy(code_hbm.at[idx_v], rows_v, sem).wait()
        pltpu.sync_copy(rows_v, out_hbm.at[pl.ds(b, SC_CH)])


@functools.cache
def _sc_gather(m):
    mesh = plsc.VectorSubcoreMesh(core_axis_name="c", subcore_axis_name="s")
    return pl.kernel(
        functools.partial(_sc_gather_body, m // SC_NW),
        out_type=jax.ShapeDtypeStruct((m, D), jnp.float32),
        mesh=mesh,
        scratch_types=[
            pltpu.VMEM((SC_CH,), jnp.int32),
            pltpu.VMEM((SC_CH, D), jnp.float32),
            pltpu.SemaphoreType.DMA,
        ],
    )


def _prep_call(code):
    return pl.pallas_call(
        _prep_body,
        out_specs=[
            pl.BlockSpec((D, NE), lambda: (0, 0)),
            pl.BlockSpec((1, NE), lambda: (0, 0)),
        ],
        out_shape=[
            jax.ShapeDtypeStruct((D, NE), jnp.float32),
            jax.ShapeDtypeStruct((1, NE), jnp.float32),
        ],
    )(code)


def _argmin_call(z_flat, cnt, c2):
    nb = z_flat.shape[0] // BM
    return pl.pallas_call(
        _argmin_body,
        grid=(nb,),
        in_specs=[
            pl.BlockSpec((BM, D), lambda i: (i, 0)),
            pl.BlockSpec((D, NE), lambda i: (0, 0)),
            pl.BlockSpec((1, NE), lambda i: (0, 0)),
        ],
        out_specs=pl.BlockSpec((1, 1, BM), lambda i: (i, 0, 0)),
        out_shape=jax.ShapeDtypeStruct((nb, 1, BM), jnp.int32),
    )(z_flat, cnt, c2)


def _finalize_call(z_flat, gathered):
    nbc = z_flat.shape[0] // BMC
    return pl.pallas_call(
        functools.partial(_finalize_body, nbc),
        grid=(nbc,),
        in_specs=[
            pl.BlockSpec((BMC, D), lambda i: (i, 0)),
            pl.BlockSpec((BMC, D), lambda i: (i, 0)),
        ],
        out_specs=[
            pl.BlockSpec((BMC, D), lambda i: (i, 0)),
            pl.BlockSpec((1, 1), lambda i: (0, 0)),
        ],
        out_shape=[
            jax.ShapeDtypeStruct((z_flat.shape[0], D), jnp.float32),
            jax.ShapeDtypeStruct((1, 1), jnp.float32),
        ],
    )(z_flat, gathered)


def kernel(z, code):
    z_flat = z.reshape(N_TOK, D)
    cnt, c2 = _prep_call(code)
    idx = _argmin_call(z_flat, cnt, c2).reshape(N_TOK)
    gathered = _sc_gather(N_TOK)(code, idx)
    zq_flat, loss11 = _finalize_call(z_flat, gathered)
    return (zq_flat.reshape(z.shape), loss11[0, 0], (None, None, idx))


# speedup vs baseline: 1.1332x; 1.0075x over previous
import functools

import jax
import jax.numpy as jnp
from jax import lax
from jax.experimental import pallas as pl
from jax.experimental.pallas import tpu as pltpu
from jax.experimental.pallas import tpu_sc as plsc

N_TOK = 16384
NE = 8192
D = 256
BM = 512
BMC = 512

SC_NC = 2
SC_NS = 16
SC_NW = SC_NC * SC_NS
SC_CH = 128


def _prep_body(code_ref, cnt_ref, c2_ref):
    c = code_ref[...]
    n = jnp.sqrt(jnp.sum(c * c, axis=1, keepdims=True))
    cn = c / jnp.maximum(n, 1e-12)
    cnt_ref[...] = cn.T
    cnt = cnt_ref[...]
    c2_ref[...] = jnp.sum(cnt * cnt, axis=0, keepdims=True)


def _argmin_body(z_ref, cnt_ref, c2_ref, idx_ref):
    z = z_ref[...]
    zn = z / jnp.maximum(jnp.sqrt(jnp.sum(z * z, axis=1, keepdims=True)), 1e-12)
    z2n = jnp.sum(zn * zn, axis=1, keepdims=True)
    s2 = lax.dot_general(zn * (-2.0), cnt_ref[...], (((1,), (0,)), ((), ())),
                         preferred_element_type=jnp.float32)
    t = (z2n + c2_ref[...]) + s2
    idx_ref[0, 0, :] = jnp.argmin(t, axis=1).astype(jnp.int32)


def _finalize2_body(nh, z_ref, ga_ref, gb_ref, zq_ref, loss_ref):
    i = pl.program_id(0)
    g = jnp.concatenate([ga_ref[...], gb_ref[...]], axis=0)
    z = jnp.concatenate([z_ref[0], z_ref[1]], axis=0)
    n = jnp.sqrt(jnp.sum(g * g, axis=1, keepdims=True))
    zq = g / jnp.maximum(n, 1e-12)
    zq_ref[0] = zq[:BMC]
    zq_ref[1] = zq[BMC:]
    dlt = zq - z
    ssq = jnp.sum(dlt * dlt, axis=(0, 1), keepdims=True)
    tot = jnp.where(i == 0, jnp.zeros_like(ssq), loss_ref[...]) + ssq
    loss_ref[...] = jnp.where(i == nh - 1, tot * (1.25 / (N_TOK * D)), tot)


def _finalize2_call(z_flat, g_a, g_b):
    half = z_flat.shape[0] // 2
    nh = half // BMC
    z3 = z_flat.reshape(2, half, D)
    zq3, loss11 = pl.pallas_call(
        functools.partial(_finalize2_body, nh),
        grid=(nh,),
        in_specs=[
            pl.BlockSpec((2, BMC, D), lambda i: (0, i, 0)),
            pl.BlockSpec((BMC, D), lambda i: (i, 0)),
            pl.BlockSpec((BMC, D), lambda i: (i, 0)),
        ],
        out_specs=[
            pl.BlockSpec((2, BMC, D), lambda i: (0, i, 0)),
            pl.BlockSpec((1, 1), lambda i: (0, 0)),
        ],
        out_shape=[
            jax.ShapeDtypeStruct((2, half, D), jnp.float32),
            jax.ShapeDtypeStruct((1, 1), jnp.float32),
        ],
    )(z3, g_a, g_b)
    return zq3.reshape(z_flat.shape), loss11


def _finalize_body(nbc, z_ref, g_ref, zq_ref, loss_ref):
    i = pl.program_id(0)
    g = g_ref[...]
    n = jnp.sqrt(jnp.sum(g * g, axis=1, keepdims=True))
    zq = g / jnp.maximum(n, 1e-12)
    zq_ref[...] = zq
    dlt = zq - z_ref[...]
    ssq = jnp.sum(dlt * dlt, axis=(0, 1), keepdims=True)
    tot = jnp.where(i == 0, jnp.zeros_like(ssq), loss_ref[...]) + ssq
    loss_ref[...] = jnp.where(i == nbc - 1, tot * (1.25 / (N_TOK * D)), tot)


def _sc_gather_body(bpw, code_hbm, idx_hbm, out_hbm, idx_v, rows0, rows1, sem):
    wid = lax.axis_index("s") * SC_NC + lax.axis_index("c")
    base = wid * bpw
    nch = bpw // SC_CH
    rows = (rows0, rows1)
    pltpu.sync_copy(idx_hbm.at[pl.ds(base, bpw)], idx_v)

    def _start(c):
        return pltpu.async_copy(
            code_hbm.at[idx_v.at[pl.ds(c * SC_CH, SC_CH)]], rows[c % 2], sem)

    cp = _start(0)
    for c in range(nch):
        cp.wait()
        nxt = _start(c + 1) if c + 1 < nch else None
        pltpu.sync_copy(rows[c % 2], out_hbm.at[pl.ds(base + c * SC_CH, SC_CH)])
        cp = nxt


@functools.cache
def _sc_gather(m):
    mesh = plsc.VectorSubcoreMesh(core_axis_name="c", subcore_axis_name="s")
    return pl.kernel(
        functools.partial(_sc_gather_body, m // SC_NW),
        out_type=jax.ShapeDtypeStruct((m, D), jnp.float32),
        mesh=mesh,
        scratch_types=[
            pltpu.VMEM((m // SC_NW,), jnp.int32),
            pltpu.VMEM((SC_CH, D), jnp.float32),
            pltpu.VMEM((SC_CH, D), jnp.float32),
            pltpu.SemaphoreType.DMA,
        ],
    )


def _prep_call(code):
    return pl.pallas_call(
        _prep_body,
        out_specs=[
            pl.BlockSpec((D, NE), lambda: (0, 0)),
            pl.BlockSpec((1, NE), lambda: (0, 0)),
        ],
        out_shape=[
            jax.ShapeDtypeStruct((D, NE), jnp.float32),
            jax.ShapeDtypeStruct((1, NE), jnp.float32),
        ],
    )(code)


def _argmin_call(z_flat, cnt, c2):
    nb = z_flat.shape[0] // BM
    return pl.pallas_call(
        _argmin_body,
        grid=(nb,),
        in_specs=[
            pl.BlockSpec((BM, D), lambda i: (i, 0)),
            pl.BlockSpec((D, NE), lambda i: (0, 0)),
            pl.BlockSpec((1, NE), lambda i: (0, 0)),
        ],
        out_specs=pl.BlockSpec((1, 1, BM), lambda i: (i, 0, 0)),
        out_shape=jax.ShapeDtypeStruct((nb, 1, BM), jnp.int32),
    )(z_flat, cnt, c2)


def _finalize_call(z_flat, gathered):
    nbc = z_flat.shape[0] // BMC
    return pl.pallas_call(
        functools.partial(_finalize_body, nbc),
        grid=(nbc,),
        in_specs=[
            pl.BlockSpec((BMC, D), lambda i: (i, 0)),
            pl.BlockSpec((BMC, D), lambda i: (i, 0)),
        ],
        out_specs=[
            pl.BlockSpec((BMC, D), lambda i: (i, 0)),
            pl.BlockSpec((1, 1), lambda i: (0, 0)),
        ],
        out_shape=[
            jax.ShapeDtypeStruct((z_flat.shape[0], D), jnp.float32),
            jax.ShapeDtypeStruct((1, 1), jnp.float32),
        ],
    )(z_flat, gathered)


def kernel(z, code):
    z_flat = z.reshape(N_TOK, D)
    cnt, c2 = _prep_call(code)
    idx = _argmin_call(z_flat, cnt, c2).reshape(N_TOK)
    gathered = _sc_gather(N_TOK)(code, idx)
    zq_flat, loss11 = _finalize_call(z_flat, gathered)
    return (zq_flat.reshape(z.shape), loss11[0, 0], (None, None, idx))
